# Initial kernel scaffold; baseline (speedup 1.0000x reference)
import jax
import jax.numpy as jnp
from jax.experimental import pallas as pl


def kernel(modal_input_0, modal_input_1, modal_input_2, modal_input_3, modal_input_4, modal_input_5, proj_g_0, proj_b_0, proj_W_0, proj_bias_0, proj_g_1, proj_b_1, proj_W_1, proj_bias_1, proj_g_2, proj_b_2, proj_W_2, proj_bias_2, proj_g_3, proj_b_3, proj_W_3, proj_bias_3, proj_g_4, proj_b_4, proj_W_4, proj_bias_4, proj_g_5, proj_b_5, proj_W_5, proj_bias_5, dec_g_0, dec_b_0, dec_W1_0, dec_b1_0, dec_W2_0, dec_b2_0, dec_g_1, dec_b_1, dec_W1_1, dec_b1_1, dec_W2_1, dec_b2_1, dec_g_2, dec_b_2, dec_W1_2, dec_b1_2, dec_W2_2, dec_b2_2, dec_g_3, dec_b_3, dec_W1_3, dec_b1_3, dec_W2_3, dec_b2_3, codebooks):
    raise NotImplementedError("write your pallas kernel here")


# speedup vs baseline: 1.3401x; 1.3401x over previous
import functools
import math

import jax
import jax.numpy as jnp
from jax import lax
from jax.experimental import pallas as pl
from jax.experimental.pallas import tpu as pltpu

_PREC = None


def _layernorm(x, g, b):
    m = jnp.mean(x, axis=-1, keepdims=True)
    v = jnp.mean((x - m) ** 2, axis=-1, keepdims=True)
    return (x - m) / jnp.sqrt(v + 1e-5) * g + b


def _proj_body(x_ref, g_ref, b_ref, w_ref, bias_ref, o_ref):
    h = _layernorm(x_ref[...], g_ref[...], b_ref[...])
    h = jnp.dot(h, w_ref[...], preferred_element_type=jnp.float32,
                precision=_PREC) + bias_ref[...]
    o_ref[...] = h * jax.nn.sigmoid(h)


def _projector(x, g, b, w, bias, bt):
    n, d = x.shape
    emb = w.shape[1]
    return pl.pallas_call(
        _proj_body,
        grid=(n // bt,),
        in_specs=[
            pl.BlockSpec((bt, d), lambda i: (i, 0)),
            pl.BlockSpec((1, d), lambda i: (0, 0)),
            pl.BlockSpec((1, d), lambda i: (0, 0)),
            pl.BlockSpec((d, emb), lambda i: (0, 0)),
            pl.BlockSpec((1, emb), lambda i: (0, 0)),
        ],
        out_specs=pl.BlockSpec((bt, emb), lambda i: (i, 0)),
        out_shape=jax.ShapeDtypeStruct((n, emb), jnp.float32),
    )(x, g.reshape(1, d), b.reshape(1, d), w, bias.reshape(1, emb))


_INV_SQRT2 = 1.0 / math.sqrt(2.0)


def _dec_body(n_in, z_refs_and_w):
    (*z_refs, g_ref, b_ref, w1_ref, b1_ref, w2_ref, b2_ref, o_ref) = z_refs_and_w
    z = z_refs[0][...]
    for zr in z_refs[1:]:
        z = z + zr[...]
    h = _layernorm(z, g_ref[...], b_ref[...])
    h = jnp.dot(h, w1_ref[...], preferred_element_type=jnp.float32,
                precision=_PREC) + b1_ref[...]
    h = 0.5 * h * (1.0 + lax.erf(h * _INV_SQRT2))
    o_ref[...] = jnp.dot(h, w2_ref[...], preferred_element_type=jnp.float32,
                         precision=_PREC) + b2_ref[...]


def _decoder(zs, g, b, w1, b1, w2, b2, bt):
    n, emb = zs[0].shape
    hid = w1.shape[1]
    o = w2.shape[1]
    body = functools.partial(lambda *refs: _dec_body(len(zs), refs))
    return pl.pallas_call(
        body,
        grid=(n // bt,),
        in_specs=[pl.BlockSpec((bt, emb), lambda i: (i, 0)) for _ in zs] + [
            pl.BlockSpec((1, emb), lambda i: (0, 0)),
            pl.BlockSpec((1, emb), lambda i: (0, 0)),
            pl.BlockSpec((emb, hid), lambda i: (0, 0)),
            pl.BlockSpec((1, hid), lambda i: (0, 0)),
            pl.BlockSpec((hid, o), lambda i: (0, 0)),
            pl.BlockSpec((1, o), lambda i: (0, 0)),
        ],
        out_specs=pl.BlockSpec((bt, o), lambda i: (i, 0)),
        out_shape=jax.ShapeDtypeStruct((n, o), jnp.float32),
    )(*zs, g.reshape(1, emb), b.reshape(1, emb), w1, b1.reshape(1, hid),
      w2, b2.reshape(1, o))


def _vq_body(depth, k, scale, z_ref, cb_ref, codes_ref, vec_ref, loss_ref):
    li = pl.program_id(0)
    ti = pl.program_id(1)
    z = z_ref[0]
    r = z
    quant = jnp.zeros_like(z)
    iota = lax.broadcasted_iota(jnp.int32, (z.shape[0], k), 1)
    for d in range(depth):
        cb = cb_ref[d]
        c2 = jnp.sum(cb * cb, axis=1)[None, :]
        r2 = jnp.sum(r * r, axis=1, keepdims=True)
        cross = lax.dot_general(r, cb, (((1,), (1,)), ((), ())),
                                preferred_element_type=jnp.float32,
                                precision=_PREC)
        dist = r2 - 2.0 * cross + c2
        m = jnp.min(dist, axis=1, keepdims=True)
        idx = jnp.min(jnp.where(dist == m, iota, k), axis=1)
        onehot = (iota == idx[:, None]).astype(jnp.float32)
        q = jnp.dot(onehot, cb, preferred_element_type=jnp.float32,
                    precision=lax.Precision.HIGHEST)
        quant = quant + q
        r = r - q
        codes_ref[0, d, :] = idx
    vec_ref[0] = z + (quant - z)
    resid = z - quant
    part = jnp.sum(resid * resid) * scale
    prev = jnp.where((li == 0) & (ti == 0), 0.0, loss_ref[0, 0])
    loss_ref[0, 0] = prev + part


def _vq(lats, codebooks, bt):
    nl, n, emb = lats.shape
    depth, k, _ = codebooks.shape
    scale = 1.25 / (n * emb)
    body = functools.partial(_vq_body, depth, k, scale)
    return pl.pallas_call(
        body,
        grid=(nl, n // bt),
        in_specs=[
            pl.BlockSpec((1, bt, emb), lambda l, t: (l, t, 0)),
            pl.BlockSpec((depth, k, emb), lambda l, t: (0, 0, 0)),
        ],
        out_specs=[
            pl.BlockSpec((1, depth, bt), lambda l, t: (l, 0, t)),
            pl.BlockSpec((1, bt, emb), lambda l, t: (l, t, 0)),
            pl.BlockSpec(memory_space=pltpu.SMEM, block_shape=(1, 1),
                         index_map=lambda l, t: (0, 0)),
        ],
        out_shape=[
            jax.ShapeDtypeStruct((nl, depth, n), jnp.int32),
            jax.ShapeDtypeStruct((nl, n, emb), jnp.float32),
            jax.ShapeDtypeStruct((1, 1), jnp.float32),
        ],
    )(lats, codebooks)


def kernel(modal_input_0, modal_input_1, modal_input_2, modal_input_3,
           modal_input_4, modal_input_5,
           proj_g_0, proj_b_0, proj_W_0, proj_bias_0,
           proj_g_1, proj_b_1, proj_W_1, proj_bias_1,
           proj_g_2, proj_b_2, proj_W_2, proj_bias_2,
           proj_g_3, proj_b_3, proj_W_3, proj_bias_3,
           proj_g_4, proj_b_4, proj_W_4, proj_bias_4,
           proj_g_5, proj_b_5, proj_W_5, proj_bias_5,
           dec_g_0, dec_b_0, dec_W1_0, dec_b1_0, dec_W2_0, dec_b2_0,
           dec_g_1, dec_b_1, dec_W1_1, dec_b1_1, dec_W2_1, dec_b2_1,
           dec_g_2, dec_b_2, dec_W1_2, dec_b1_2, dec_W2_2, dec_b2_2,
           dec_g_3, dec_b_3, dec_W1_3, dec_b1_3, dec_W2_3, dec_b2_3,
           codebooks):
    xs = [modal_input_0, modal_input_1, modal_input_2, modal_input_3,
          modal_input_4, modal_input_5]
    pg = [proj_g_0, proj_g_1, proj_g_2, proj_g_3, proj_g_4, proj_g_5]
    pb = [proj_b_0, proj_b_1, proj_b_2, proj_b_3, proj_b_4, proj_b_5]
    pw = [proj_W_0, proj_W_1, proj_W_2, proj_W_3, proj_W_4, proj_W_5]
    pbias = [proj_bias_0, proj_bias_1, proj_bias_2, proj_bias_3,
             proj_bias_4, proj_bias_5]
    n = xs[0].shape[0]
    bt = 256 if n % 256 == 0 else n

    latents = [_projector(xs[i], pg[i], pb[i], pw[i], pbias[i], bt)
               for i in range(6)]

    dg = [dec_g_0, dec_g_1, dec_g_2, dec_g_3]
    db = [dec_b_0, dec_b_1, dec_b_2, dec_b_3]
    dw1 = [dec_W1_0, dec_W1_1, dec_W1_2, dec_W1_3]
    db1 = [dec_b1_0, dec_b1_1, dec_b1_2, dec_b1_3]
    dw2 = [dec_W2_0, dec_W2_1, dec_W2_2, dec_W2_3]
    db2 = [dec_b2_0, dec_b2_1, dec_b2_2, dec_b2_3]
    dec_in = [latents[:3], [latents[3]], [latents[4]], [latents[5]]]
    recons = [_decoder(dec_in[j], dg[j], db[j], dw1[j], db1[j],
                       dw2[j], db2[j], bt)
              for j in range(4)]

    modal_latents = jnp.stack(latents, axis=0)
    latent_codes, latent_vectors, loss = _vq(modal_latents, codebooks, bt)
    vq_loss = loss.reshape(())
    return (latent_codes, latent_vectors, vq_loss, modal_latents,
            recons[0], recons[1], recons[2], recons[3])


# speedup vs baseline: 1.4340x; 1.0701x over previous
import functools
import math

import jax
import jax.numpy as jnp
from jax import lax
from jax.experimental import pallas as pl
from jax.experimental.pallas import tpu as pltpu

_PREC = None


def _layernorm(x, g, b):
    m = jnp.mean(x, axis=-1, keepdims=True)
    v = jnp.mean((x - m) ** 2, axis=-1, keepdims=True)
    return (x - m) / jnp.sqrt(v + 1e-5) * g + b


def _proj_body(x_ref, g_ref, b_ref, w_ref, bias_ref, o_ref):
    h = _layernorm(x_ref[...], g_ref[...], b_ref[...])
    h = jnp.dot(h, w_ref[...], preferred_element_type=jnp.float32,
                precision=_PREC) + bias_ref[...]
    o_ref[...] = h * jax.nn.sigmoid(h)


def _projector(x, g, b, w, bias, bt):
    n, d = x.shape
    emb = w.shape[1]
    return pl.pallas_call(
        _proj_body,
        grid=(n // bt,),
        in_specs=[
            pl.BlockSpec((bt, d), lambda i: (i, 0)),
            pl.BlockSpec((1, d), lambda i: (0, 0)),
            pl.BlockSpec((1, d), lambda i: (0, 0)),
            pl.BlockSpec((d, emb), lambda i: (0, 0)),
            pl.BlockSpec((1, emb), lambda i: (0, 0)),
        ],
        out_specs=pl.BlockSpec((bt, emb), lambda i: (i, 0)),
        out_shape=jax.ShapeDtypeStruct((n, emb), jnp.float32),
    )(x, g.reshape(1, d), b.reshape(1, d), w, bias.reshape(1, emb))


_INV_SQRT2 = 1.0 / math.sqrt(2.0)


def _dec_body(n_in, z_refs_and_w):
    (*z_refs, g_ref, b_ref, w1_ref, b1_ref, w2_ref, b2_ref, o_ref) = z_refs_and_w
    z = z_refs[0][...]
    for zr in z_refs[1:]:
        z = z + zr[...]
    h = _layernorm(z, g_ref[...], b_ref[...])
    h = jnp.dot(h, w1_ref[...], preferred_element_type=jnp.float32,
                precision=_PREC) + b1_ref[...]
    h = 0.5 * h * (1.0 + lax.erf(h * _INV_SQRT2))
    o_ref[...] = jnp.dot(h, w2_ref[...], preferred_element_type=jnp.float32,
                         precision=_PREC) + b2_ref[...]


def _decoder(zs, g, b, w1, b1, w2, b2, bt):
    n, emb = zs[0].shape
    hid = w1.shape[1]
    o = w2.shape[1]
    body = functools.partial(lambda *refs: _dec_body(len(zs), refs))
    return pl.pallas_call(
        body,
        grid=(n // bt,),
        in_specs=[pl.BlockSpec((bt, emb), lambda i: (i, 0)) for _ in zs] + [
            pl.BlockSpec((1, emb), lambda i: (0, 0)),
            pl.BlockSpec((1, emb), lambda i: (0, 0)),
            pl.BlockSpec((emb, hid), lambda i: (0, 0)),
            pl.BlockSpec((1, hid), lambda i: (0, 0)),
            pl.BlockSpec((hid, o), lambda i: (0, 0)),
            pl.BlockSpec((1, o), lambda i: (0, 0)),
        ],
        out_specs=pl.BlockSpec((bt, o), lambda i: (i, 0)),
        out_shape=jax.ShapeDtypeStruct((n, o), jnp.float32),
    )(*zs, g.reshape(1, emb), b.reshape(1, emb), w1, b1.reshape(1, hid),
      w2, b2.reshape(1, o))


def _vq_body(depth, k, scale, z_ref, cb_ref, by0_ref, by1_ref, by2_ref,
             by3_ref, codes_ref, vec_ref, loss_ref):
    z = z_ref[0]
    r = z
    quant = jnp.zeros_like(z)
    iota = lax.broadcasted_iota(jnp.int32, (z.shape[0], k), 1)
    for d in range(depth):
        cb = cb_ref[d]
        c2 = jnp.sum(cb * cb, axis=1)[None, :]
        r2 = jnp.sum(r * r, axis=1, keepdims=True)
        cross = lax.dot_general(r, cb, (((1,), (1,)), ((), ())),
                                preferred_element_type=jnp.float32,
                                precision=_PREC)
        dist = r2 - 2.0 * cross + c2
        m = jnp.min(dist, axis=1, keepdims=True)
        idx = jnp.min(jnp.where(dist == m, iota, k), axis=1)
        oh8 = (iota == idx[:, None]).astype(jnp.int8)
        b0 = jnp.dot(oh8, by0_ref[d], preferred_element_type=jnp.int32)
        b1 = jnp.dot(oh8, by1_ref[d], preferred_element_type=jnp.int32)
        b2 = jnp.dot(oh8, by2_ref[d], preferred_element_type=jnp.int32)
        b3 = jnp.dot(oh8, by3_ref[d], preferred_element_type=jnp.int32)
        word = ((b3 & 0xFF) << 24) | ((b2 & 0xFF) << 16) \
               | ((b1 & 0xFF) << 8) | (b0 & 0xFF)
        q = lax.bitcast_convert_type(word, jnp.float32)
        quant = quant + q
        r = r - q
        codes_ref[0, d, :] = idx
    vec_ref[0] = z + (quant - z)
    resid = z - quant
    loss_ref[...] = jnp.broadcast_to(jnp.sum(resid * resid) * scale,
                                     loss_ref.shape)


def _vq(lats, codebooks, bt):
    nl, n, emb = lats.shape
    depth, k, _ = codebooks.shape
    scale = 1.25 / (n * emb)
    w = lax.bitcast_convert_type(codebooks, jnp.int32)
    by = [((w >> (8 * i)) & 0xFF).astype(jnp.int8) for i in range(4)]
    body = functools.partial(_vq_body, depth, k, scale)
    cb_spec = pl.BlockSpec((depth, k, emb), lambda l, t: (0, 0, 0))
    return pl.pallas_call(
        body,
        grid=(nl, n // bt),
        in_specs=[
            pl.BlockSpec((1, bt, emb), lambda l, t: (l, t, 0)),
            cb_spec, cb_spec, cb_spec, cb_spec, cb_spec,
        ],
        out_specs=[
            pl.BlockSpec((1, depth, bt), lambda l, t: (l, 0, t)),
            pl.BlockSpec((1, bt, emb), lambda l, t: (l, t, 0)),
            pl.BlockSpec((1, 1, 128), lambda l, t: (l, 0, t)),
        ],
        out_shape=[
            jax.ShapeDtypeStruct((nl, depth, n), jnp.int32),
            jax.ShapeDtypeStruct((nl, n, emb), jnp.float32),
            jax.ShapeDtypeStruct((nl, 1, (n // bt) * 128), jnp.float32),
        ],
    )(lats, codebooks, *by)


def kernel(modal_input_0, modal_input_1, modal_input_2, modal_input_3,
           modal_input_4, modal_input_5,
           proj_g_0, proj_b_0, proj_W_0, proj_bias_0,
           proj_g_1, proj_b_1, proj_W_1, proj_bias_1,
           proj_g_2, proj_b_2, proj_W_2, proj_bias_2,
           proj_g_3, proj_b_3, proj_W_3, proj_bias_3,
           proj_g_4, proj_b_4, proj_W_4, proj_bias_4,
           proj_g_5, proj_b_5, proj_W_5, proj_bias_5,
           dec_g_0, dec_b_0, dec_W1_0, dec_b1_0, dec_W2_0, dec_b2_0,
           dec_g_1, dec_b_1, dec_W1_1, dec_b1_1, dec_W2_1, dec_b2_1,
           dec_g_2, dec_b_2, dec_W1_2, dec_b1_2, dec_W2_2, dec_b2_2,
           dec_g_3, dec_b_3, dec_W1_3, dec_b1_3, dec_W2_3, dec_b2_3,
           codebooks):
    xs = [modal_input_0, modal_input_1, modal_input_2, modal_input_3,
          modal_input_4, modal_input_5]
    pg = [proj_g_0, proj_g_1, proj_g_2, proj_g_3, proj_g_4, proj_g_5]
    pb = [proj_b_0, proj_b_1, proj_b_2, proj_b_3, proj_b_4, proj_b_5]
    pw = [proj_W_0, proj_W_1, proj_W_2, proj_W_3, proj_W_4, proj_W_5]
    pbias = [proj_bias_0, proj_bias_1, proj_bias_2, proj_bias_3,
             proj_bias_4, proj_bias_5]
    n = xs[0].shape[0]
    bt = 256 if n % 256 == 0 else n

    latents = [_projector(xs[i], pg[i], pb[i], pw[i], pbias[i], bt)
               for i in range(6)]

    dg = [dec_g_0, dec_g_1, dec_g_2, dec_g_3]
    db = [dec_b_0, dec_b_1, dec_b_2, dec_b_3]
    dw1 = [dec_W1_0, dec_W1_1, dec_W1_2, dec_W1_3]
    db1 = [dec_b1_0, dec_b1_1, dec_b1_2, dec_b1_3]
    dw2 = [dec_W2_0, dec_W2_1, dec_W2_2, dec_W2_3]
    db2 = [dec_b2_0, dec_b2_1, dec_b2_2, dec_b2_3]
    dec_in = [latents[:3], [latents[3]], [latents[4]], [latents[5]]]
    recons = [_decoder(dec_in[j], dg[j], db[j], dw1[j], db1[j],
                       dw2[j], db2[j], bt)
              for j in range(4)]

    modal_latents = jnp.stack(latents, axis=0)
    latent_codes, latent_vectors, loss = _vq(modal_latents, codebooks, bt)
    vq_loss = jnp.sum(loss[:, 0, ::128])
    return (latent_codes, latent_vectors, vq_loss, modal_latents,
            recons[0], recons[1], recons[2], recons[3])


# speedup vs baseline: 1.4402x; 1.0043x over previous
import functools
import math

import jax
import jax.numpy as jnp
from jax import lax
from jax.experimental import pallas as pl
from jax.experimental.pallas import tpu as pltpu

_PREC = None


def _layernorm(x, g, b):
    m = jnp.mean(x, axis=-1, keepdims=True)
    v = jnp.mean((x - m) ** 2, axis=-1, keepdims=True)
    return (x - m) / jnp.sqrt(v + 1e-5) * g + b


def _proj_body(x_ref, g_ref, b_ref, w_ref, bias_ref, o_ref):
    h = _layernorm(x_ref[...], g_ref[...], b_ref[...])
    h = jnp.dot(h, w_ref[...], preferred_element_type=jnp.float32,
                precision=_PREC) + bias_ref[...]
    o_ref[...] = h * jax.nn.sigmoid(h)


def _projector(x, g, b, w, bias, bt):
    n, d = x.shape
    emb = w.shape[1]
    return pl.pallas_call(
        _proj_body,
        grid=(n // bt,),
        in_specs=[
            pl.BlockSpec((bt, d), lambda i: (i, 0)),
            pl.BlockSpec((1, d), lambda i: (0, 0)),
            pl.BlockSpec((1, d), lambda i: (0, 0)),
            pl.BlockSpec((d, emb), lambda i: (0, 0)),
            pl.BlockSpec((1, emb), lambda i: (0, 0)),
        ],
        out_specs=pl.BlockSpec((bt, emb), lambda i: (i, 0)),
        out_shape=jax.ShapeDtypeStruct((n, emb), jnp.float32),
    )(x, g.reshape(1, d), b.reshape(1, d), w, bias.reshape(1, emb))


_INV_SQRT2 = 1.0 / math.sqrt(2.0)


def _dec_body(n_in, z_refs_and_w):
    (*z_refs, g_ref, b_ref, w1_ref, b1_ref, w2_ref, b2_ref, o_ref) = z_refs_and_w
    z = z_refs[0][...]
    for zr in z_refs[1:]:
        z = z + zr[...]
    h = _layernorm(z, g_ref[...], b_ref[...])
    h = jnp.dot(h, w1_ref[...], preferred_element_type=jnp.float32,
                precision=_PREC) + b1_ref[...]
    h = 0.5 * h * (1.0 + lax.erf(h * _INV_SQRT2))
    o_ref[...] = jnp.dot(h, w2_ref[...], preferred_element_type=jnp.float32,
                         precision=_PREC) + b2_ref[...]


def _decoder(zs, g, b, w1, b1, w2, b2, bt):
    n, emb = zs[0].shape
    hid = w1.shape[1]
    o = w2.shape[1]
    body = functools.partial(lambda *refs: _dec_body(len(zs), refs))
    return pl.pallas_call(
        body,
        grid=(n // bt,),
        in_specs=[pl.BlockSpec((bt, emb), lambda i: (i, 0)) for _ in zs] + [
            pl.BlockSpec((1, emb), lambda i: (0, 0)),
            pl.BlockSpec((1, emb), lambda i: (0, 0)),
            pl.BlockSpec((emb, hid), lambda i: (0, 0)),
            pl.BlockSpec((1, hid), lambda i: (0, 0)),
            pl.BlockSpec((hid, o), lambda i: (0, 0)),
            pl.BlockSpec((1, o), lambda i: (0, 0)),
        ],
        out_specs=pl.BlockSpec((bt, o), lambda i: (i, 0)),
        out_shape=jax.ShapeDtypeStruct((n, o), jnp.float32),
    )(*zs, g.reshape(1, emb), b.reshape(1, emb), w1, b1.reshape(1, hid),
      w2, b2.reshape(1, o))


def _vq_body(depth, k, scale, z_ref, cb_ref, by0_ref, by1_ref, by2_ref,
             by3_ref, codes_ref, vec_ref, loss_ref, c2_ref):
    z = z_ref[0]
    r = z
    quant = jnp.zeros_like(z)
    iota = lax.broadcasted_iota(jnp.int32, (z.shape[0], k), 1)

    first = (pl.program_id(0) == 0) & (pl.program_id(1) == 0)

    @pl.when(first)
    def _():
        c2_ref[...] = jnp.sum(cb_ref[...] * cb_ref[...], axis=2)

    for d in range(depth):
        cb = cb_ref[d]
        c2 = c2_ref[d][None, :]
        r2 = jnp.sum(r * r, axis=1, keepdims=True)
        cross = lax.dot_general(r, cb, (((1,), (1,)), ((), ())),
                                preferred_element_type=jnp.float32,
                                precision=_PREC)
        dist = r2 - 2.0 * cross + c2
        idx = jnp.argmin(dist, axis=1).astype(jnp.int32)
        ohb = (iota == idx[:, None]).astype(jnp.bfloat16)
        b0 = jnp.dot(ohb, by0_ref[d], preferred_element_type=jnp.float32)
        b1 = jnp.dot(ohb, by1_ref[d], preferred_element_type=jnp.float32)
        b2 = jnp.dot(ohb, by2_ref[d], preferred_element_type=jnp.float32)
        b3 = jnp.dot(ohb, by3_ref[d], preferred_element_type=jnp.float32)
        hi = (b3 * 256.0 + b2).astype(jnp.int32)
        lo = (b1 * 256.0 + b0).astype(jnp.int32)
        q = lax.bitcast_convert_type((hi << 16) | lo, jnp.float32)
        quant = quant + q
        r = r - q
        codes_ref[0, d, :] = idx
    vec_ref[0] = z + (quant - z)
    resid = z - quant
    loss_ref[...] = jnp.broadcast_to(jnp.sum(resid * resid) * scale,
                                     loss_ref.shape)


def _vq(lats, codebooks, bt):
    nl, n, emb = lats.shape
    depth, k, _ = codebooks.shape
    scale = 1.25 / (n * emb)
    w = lax.bitcast_convert_type(codebooks, jnp.int32)
    by = [((w >> (8 * i)) & 0xFF).astype(jnp.bfloat16) for i in range(4)]
    body = functools.partial(_vq_body, depth, k, scale)
    cb_spec = pl.BlockSpec((depth, k, emb), lambda l, t: (0, 0, 0))
    return pl.pallas_call(
        body,
        grid=(nl, n // bt),
        in_specs=[
            pl.BlockSpec((1, bt, emb), lambda l, t: (l, t, 0)),
            cb_spec, cb_spec, cb_spec, cb_spec, cb_spec,
        ],
        out_specs=[
            pl.BlockSpec((1, depth, bt), lambda l, t: (l, 0, t)),
            pl.BlockSpec((1, bt, emb), lambda l, t: (l, t, 0)),
            pl.BlockSpec((1, 1, 128), lambda l, t: (l, 0, t)),
        ],
        out_shape=[
            jax.ShapeDtypeStruct((nl, depth, n), jnp.int32),
            jax.ShapeDtypeStruct((nl, n, emb), jnp.float32),
            jax.ShapeDtypeStruct((nl, 1, (n // bt) * 128), jnp.float32),
        ],
        scratch_shapes=[pltpu.VMEM((depth, k), jnp.float32)],
    )(lats, codebooks, *by)


def kernel(modal_input_0, modal_input_1, modal_input_2, modal_input_3,
           modal_input_4, modal_input_5,
           proj_g_0, proj_b_0, proj_W_0, proj_bias_0,
           proj_g_1, proj_b_1, proj_W_1, proj_bias_1,
           proj_g_2, proj_b_2, proj_W_2, proj_bias_2,
           proj_g_3, proj_b_3, proj_W_3, proj_bias_3,
           proj_g_4, proj_b_4, proj_W_4, proj_bias_4,
           proj_g_5, proj_b_5, proj_W_5, proj_bias_5,
           dec_g_0, dec_b_0, dec_W1_0, dec_b1_0, dec_W2_0, dec_b2_0,
           dec_g_1, dec_b_1, dec_W1_1, dec_b1_1, dec_W2_1, dec_b2_1,
           dec_g_2, dec_b_2, dec_W1_2, dec_b1_2, dec_W2_2, dec_b2_2,
           dec_g_3, dec_b_3, dec_W1_3, dec_b1_3, dec_W2_3, dec_b2_3,
           codebooks):
    xs = [modal_input_0, modal_input_1, modal_input_2, modal_input_3,
          modal_input_4, modal_input_5]
    pg = [proj_g_0, proj_g_1, proj_g_2, proj_g_3, proj_g_4, proj_g_5]
    pb = [proj_b_0, proj_b_1, proj_b_2, proj_b_3, proj_b_4, proj_b_5]
    pw = [proj_W_0, proj_W_1, proj_W_2, proj_W_3, proj_W_4, proj_W_5]
    pbias = [proj_bias_0, proj_bias_1, proj_bias_2, proj_bias_3,
             proj_bias_4, proj_bias_5]
    n = xs[0].shape[0]
    bt = 256 if n % 256 == 0 else n

    latents = [_projector(xs[i], pg[i], pb[i], pw[i], pbias[i], bt)
               for i in range(6)]

    dg = [dec_g_0, dec_g_1, dec_g_2, dec_g_3]
    db = [dec_b_0, dec_b_1, dec_b_2, dec_b_3]
    dw1 = [dec_W1_0, dec_W1_1, dec_W1_2, dec_W1_3]
    db1 = [dec_b1_0, dec_b1_1, dec_b1_2, dec_b1_3]
    dw2 = [dec_W2_0, dec_W2_1, dec_W2_2, dec_W2_3]
    db2 = [dec_b2_0, dec_b2_1, dec_b2_2, dec_b2_3]
    dec_in = [latents[:3], [latents[3]], [latents[4]], [latents[5]]]
    recons = [_decoder(dec_in[j], dg[j], db[j], dw1[j], db1[j],
                       dw2[j], db2[j], bt)
              for j in range(4)]

    modal_latents = jnp.stack(latents, axis=0)
    latent_codes, latent_vectors, loss = _vq(modal_latents, codebooks, bt)
    vq_loss = jnp.sum(loss[:, 0, ::128])
    return (latent_codes, latent_vectors, vq_loss, modal_latents,
            recons[0], recons[1], recons[2], recons[3])


# speedup vs baseline: 1.5147x; 1.0517x over previous
import functools
import math

import jax
import jax.numpy as jnp
from jax import lax
from jax.experimental import pallas as pl
from jax.experimental.pallas import tpu as pltpu

_PREC = None


def _layernorm(x, g, b):
    m = jnp.mean(x, axis=-1, keepdims=True)
    v = jnp.mean((x - m) ** 2, axis=-1, keepdims=True)
    return (x - m) / jnp.sqrt(v + 1e-5) * g + b


def _proj_body(nmod, refs):
    o_ref = refs[-1]
    for i in range(nmod):
        x_ref, g_ref, b_ref, w_ref, bias_ref = refs[5 * i:5 * i + 5]
        h = _layernorm(x_ref[...], g_ref[...], b_ref[...])
        h = jnp.dot(h, w_ref[...], preferred_element_type=jnp.float32,
                    precision=_PREC) + bias_ref[...]
        o_ref[i] = h * jax.nn.sigmoid(h)


def _projectors(xs, gs, bs, ws, biases, bt):
    n = xs[0].shape[0]
    emb = ws[0].shape[1]
    nmod = len(xs)
    in_specs = []
    args = []
    for x, g, b, w, bias in zip(xs, gs, bs, ws, biases):
        d = x.shape[1]
        in_specs += [
            pl.BlockSpec((bt, d), lambda i: (i, 0)),
            pl.BlockSpec((1, d), lambda i: (0, 0)),
            pl.BlockSpec((1, d), lambda i: (0, 0)),
            pl.BlockSpec((d, emb), lambda i: (0, 0)),
            pl.BlockSpec((1, emb), lambda i: (0, 0)),
        ]
        args += [x, g.reshape(1, d), b.reshape(1, d), w, bias.reshape(1, emb)]
    return pl.pallas_call(
        functools.partial(lambda nm, *refs: _proj_body(nm, refs), nmod),
        grid=(n // bt,),
        in_specs=in_specs,
        out_specs=pl.BlockSpec((nmod, bt, emb), lambda i: (0, i, 0)),
        out_shape=jax.ShapeDtypeStruct((nmod, n, emb), jnp.float32),
    )(*args)


_INV_SQRT2 = 1.0 / math.sqrt(2.0)


def _dec_body(n_in, z_refs_and_w):
    (*z_refs, g_ref, b_ref, w1_ref, b1_ref, w2_ref, b2_ref, o_ref) = z_refs_and_w
    z = z_refs[0][...]
    for zr in z_refs[1:]:
        z = z + zr[...]
    h = _layernorm(z, g_ref[...], b_ref[...])
    h = jnp.dot(h, w1_ref[...], preferred_element_type=jnp.float32,
                precision=_PREC) + b1_ref[...]
    h = 0.5 * h * (1.0 + lax.erf(h * _INV_SQRT2))
    o_ref[...] = jnp.dot(h, w2_ref[...], preferred_element_type=jnp.float32,
                         precision=_PREC) + b2_ref[...]


def _decoder(zs, g, b, w1, b1, w2, b2, bt):
    n, emb = zs[0].shape
    hid = w1.shape[1]
    o = w2.shape[1]
    body = functools.partial(lambda *refs: _dec_body(len(zs), refs))
    return pl.pallas_call(
        body,
        grid=(n // bt,),
        in_specs=[pl.BlockSpec((bt, emb), lambda i: (i, 0)) for _ in zs] + [
            pl.BlockSpec((1, emb), lambda i: (0, 0)),
            pl.BlockSpec((1, emb), lambda i: (0, 0)),
            pl.BlockSpec((emb, hid), lambda i: (0, 0)),
            pl.BlockSpec((1, hid), lambda i: (0, 0)),
            pl.BlockSpec((hid, o), lambda i: (0, 0)),
            pl.BlockSpec((1, o), lambda i: (0, 0)),
        ],
        out_specs=pl.BlockSpec((bt, o), lambda i: (i, 0)),
        out_shape=jax.ShapeDtypeStruct((n, o), jnp.float32),
    )(*zs, g.reshape(1, emb), b.reshape(1, emb), w1, b1.reshape(1, hid),
      w2, b2.reshape(1, o))


def _vq_body(depth, k, scale, z_ref, cb_ref, by0_ref, by1_ref, by2_ref,
             by3_ref, codes_ref, vec_ref, loss_ref, c2_ref):
    z = z_ref[0]
    r = z
    quant = jnp.zeros_like(z)
    iota = lax.broadcasted_iota(jnp.int32, (z.shape[0], k), 1)

    first = (pl.program_id(0) == 0) & (pl.program_id(1) == 0)

    @pl.when(first)
    def _():
        c2_ref[...] = jnp.sum(cb_ref[...] * cb_ref[...], axis=2)

    for d in range(depth):
        cb = cb_ref[d]
        c2 = c2_ref[d][None, :]
        r2 = jnp.sum(r * r, axis=1, keepdims=True)
        cross = lax.dot_general(r, cb, (((1,), (1,)), ((), ())),
                                preferred_element_type=jnp.float32,
                                precision=_PREC)
        dist = r2 - 2.0 * cross + c2
        idx = jnp.argmin(dist, axis=1).astype(jnp.int32)
        ohb = (iota == idx[:, None]).astype(jnp.bfloat16)
        b0 = jnp.dot(ohb, by0_ref[d], preferred_element_type=jnp.float32)
        b1 = jnp.dot(ohb, by1_ref[d], preferred_element_type=jnp.float32)
        b2 = jnp.dot(ohb, by2_ref[d], preferred_element_type=jnp.float32)
        b3 = jnp.dot(ohb, by3_ref[d], preferred_element_type=jnp.float32)
        hi = (b3 * 256.0 + b2).astype(jnp.int32)
        lo = (b1 * 256.0 + b0).astype(jnp.int32)
        q = lax.bitcast_convert_type((hi << 16) | lo, jnp.float32)
        quant = quant + q
        r = r - q
        codes_ref[0, d, :] = idx
    vec_ref[0] = z + (quant - z)
    resid = z - quant
    loss_ref[...] = jnp.broadcast_to(jnp.sum(resid * resid) * scale,
                                     loss_ref.shape)


def _vq(lats, codebooks, bt):
    nl, n, emb = lats.shape
    depth, k, _ = codebooks.shape
    scale = 1.25 / (n * emb)
    w = lax.bitcast_convert_type(codebooks, jnp.int32)
    by = [((w >> (8 * i)) & 0xFF).astype(jnp.bfloat16) for i in range(4)]
    body = functools.partial(_vq_body, depth, k, scale)
    cb_spec = pl.BlockSpec((depth, k, emb), lambda l, t: (0, 0, 0))
    return pl.pallas_call(
        body,
        grid=(nl, n // bt),
        in_specs=[
            pl.BlockSpec((1, bt, emb), lambda l, t: (l, t, 0)),
            cb_spec, cb_spec, cb_spec, cb_spec, cb_spec,
        ],
        out_specs=[
            pl.BlockSpec((1, depth, bt), lambda l, t: (l, 0, t)),
            pl.BlockSpec((1, bt, emb), lambda l, t: (l, t, 0)),
            pl.BlockSpec((1, 1, 128), lambda l, t: (l, 0, t)),
        ],
        out_shape=[
            jax.ShapeDtypeStruct((nl, depth, n), jnp.int32),
            jax.ShapeDtypeStruct((nl, n, emb), jnp.float32),
            jax.ShapeDtypeStruct((nl, 1, (n // bt) * 128), jnp.float32),
        ],
        scratch_shapes=[pltpu.VMEM((depth, k), jnp.float32)],
    )(lats, codebooks, *by)


def kernel(modal_input_0, modal_input_1, modal_input_2, modal_input_3,
           modal_input_4, modal_input_5,
           proj_g_0, proj_b_0, proj_W_0, proj_bias_0,
           proj_g_1, proj_b_1, proj_W_1, proj_bias_1,
           proj_g_2, proj_b_2, proj_W_2, proj_bias_2,
           proj_g_3, proj_b_3, proj_W_3, proj_bias_3,
           proj_g_4, proj_b_4, proj_W_4, proj_bias_4,
           proj_g_5, proj_b_5, proj_W_5, proj_bias_5,
           dec_g_0, dec_b_0, dec_W1_0, dec_b1_0, dec_W2_0, dec_b2_0,
           dec_g_1, dec_b_1, dec_W1_1, dec_b1_1, dec_W2_1, dec_b2_1,
           dec_g_2, dec_b_2, dec_W1_2, dec_b1_2, dec_W2_2, dec_b2_2,
           dec_g_3, dec_b_3, dec_W1_3, dec_b1_3, dec_W2_3, dec_b2_3,
           codebooks):
    xs = [modal_input_0, modal_input_1, modal_input_2, modal_input_3,
          modal_input_4, modal_input_5]
    pg = [proj_g_0, proj_g_1, proj_g_2, proj_g_3, proj_g_4, proj_g_5]
    pb = [proj_b_0, proj_b_1, proj_b_2, proj_b_3, proj_b_4, proj_b_5]
    pw = [proj_W_0, proj_W_1, proj_W_2, proj_W_3, proj_W_4, proj_W_5]
    pbias = [proj_bias_0, proj_bias_1, proj_bias_2, proj_bias_3,
             proj_bias_4, proj_bias_5]
    n = xs[0].shape[0]
    bt = 256 if n % 256 == 0 else n

    modal_latents = _projectors(xs, pg, pb, pw, pbias, bt)
    latents = [modal_latents[i] for i in range(6)]

    dg = [dec_g_0, dec_g_1, dec_g_2, dec_g_3]
    db = [dec_b_0, dec_b_1, dec_b_2, dec_b_3]
    dw1 = [dec_W1_0, dec_W1_1, dec_W1_2, dec_W1_3]
    db1 = [dec_b1_0, dec_b1_1, dec_b1_2, dec_b1_3]
    dw2 = [dec_W2_0, dec_W2_1, dec_W2_2, dec_W2_3]
    db2 = [dec_b2_0, dec_b2_1, dec_b2_2, dec_b2_3]
    dec_in = [latents[:3], [latents[3]], [latents[4]], [latents[5]]]
    recons = [_decoder(dec_in[j], dg[j], db[j], dw1[j], db1[j],
                       dw2[j], db2[j], bt)
              for j in range(4)]

    latent_codes, latent_vectors, loss = _vq(modal_latents, codebooks, bt)
    vq_loss = jnp.sum(loss[:, 0, ::128])
    return (latent_codes, latent_vectors, vq_loss, modal_latents,
            recons[0], recons[1], recons[2], recons[3])


# speedup vs baseline: 1.6667x; 1.1004x over previous
import functools
import math

import jax
import jax.numpy as jnp
from jax import lax
from jax.experimental import pallas as pl
from jax.experimental.pallas import tpu as pltpu

_PREC = None


def _layernorm(x, g, b):
    m = jnp.mean(x, axis=-1, keepdims=True)
    v = jnp.mean((x - m) ** 2, axis=-1, keepdims=True)
    return (x - m) / jnp.sqrt(v + 1e-5) * g + b


def _proj_body(nmod, refs):
    o_ref = refs[-1]
    for i in range(nmod):
        x_ref, g_ref, b_ref, w_ref, bias_ref = refs[5 * i:5 * i + 5]
        h = _layernorm(x_ref[...], g_ref[...], b_ref[...])
        h = jnp.dot(h, w_ref[...], preferred_element_type=jnp.float32,
                    precision=_PREC) + bias_ref[...]
        o_ref[i] = h * jax.nn.sigmoid(h)


def _projectors(xs, gs, bs, ws, biases, bt):
    n = xs[0].shape[0]
    emb = ws[0].shape[1]
    nmod = len(xs)
    in_specs = []
    args = []
    for x, g, b, w, bias in zip(xs, gs, bs, ws, biases):
        d = x.shape[1]
        in_specs += [
            pl.BlockSpec((bt, d), lambda i: (i, 0)),
            pl.BlockSpec((1, d), lambda i: (0, 0)),
            pl.BlockSpec((1, d), lambda i: (0, 0)),
            pl.BlockSpec((d, emb), lambda i: (0, 0)),
            pl.BlockSpec((1, emb), lambda i: (0, 0)),
        ]
        args += [x, g.reshape(1, d), b.reshape(1, d), w, bias.reshape(1, emb)]
    return pl.pallas_call(
        functools.partial(lambda nm, *refs: _proj_body(nm, refs), nmod),
        grid=(n // bt,),
        in_specs=in_specs,
        out_specs=pl.BlockSpec((nmod, bt, emb), lambda i: (0, i, 0)),
        out_shape=jax.ShapeDtypeStruct((nmod, n, emb), jnp.float32),
    )(*args)


_INV_SQRT2 = 1.0 / math.sqrt(2.0)


def _dec_body(n_in, z_refs_and_w):
    (*z_refs, g_ref, b_ref, w1_ref, b1_ref, w2_ref, b2_ref, o_ref) = z_refs_and_w
    z = z_refs[0][...]
    for zr in z_refs[1:]:
        z = z + zr[...]
    h = _layernorm(z, g_ref[...], b_ref[...])
    h = jnp.dot(h, w1_ref[...], preferred_element_type=jnp.float32,
                precision=_PREC) + b1_ref[...]
    h = 0.5 * h * (1.0 + lax.erf(h * _INV_SQRT2))
    o_ref[...] = jnp.dot(h, w2_ref[...], preferred_element_type=jnp.float32,
                         precision=_PREC) + b2_ref[...]


def _decoder(zs, g, b, w1, b1, w2, b2, bt):
    n, emb = zs[0].shape
    hid = w1.shape[1]
    o = w2.shape[1]
    body = functools.partial(lambda *refs: _dec_body(len(zs), refs))
    return pl.pallas_call(
        body,
        grid=(n // bt,),
        in_specs=[pl.BlockSpec((bt, emb), lambda i: (i, 0)) for _ in zs] + [
            pl.BlockSpec((1, emb), lambda i: (0, 0)),
            pl.BlockSpec((1, emb), lambda i: (0, 0)),
            pl.BlockSpec((emb, hid), lambda i: (0, 0)),
            pl.BlockSpec((1, hid), lambda i: (0, 0)),
            pl.BlockSpec((hid, o), lambda i: (0, 0)),
            pl.BlockSpec((1, o), lambda i: (0, 0)),
        ],
        out_specs=pl.BlockSpec((bt, o), lambda i: (i, 0)),
        out_shape=jax.ShapeDtypeStruct((n, o), jnp.float32),
    )(*zs, g.reshape(1, emb), b.reshape(1, emb), w1, b1.reshape(1, hid),
      w2, b2.reshape(1, o))


def _vq_body(depth, k, scale, z_ref, cb_ref, by0_ref, by1_ref, by2_ref,
             by3_ref, codes_ref, vec_ref, loss_ref, c2_ref):
    z = z_ref[0]
    r = z
    quant = jnp.zeros_like(z)
    iota = lax.broadcasted_iota(jnp.int32, (z.shape[0], k), 1)

    first = (pl.program_id(0) == 0) & (pl.program_id(1) == 0)

    @pl.when(first)
    def _():
        c2_ref[...] = jnp.sum(cb_ref[...] * cb_ref[...], axis=2)

    for d in range(depth):
        cb = cb_ref[d]
        c2 = c2_ref[d][None, :]
        r2 = jnp.sum(r * r, axis=1, keepdims=True)
        cross = lax.dot_general(r, cb, (((1,), (1,)), ((), ())),
                                preferred_element_type=jnp.float32,
                                precision=_PREC)
        dist = r2 - 2.0 * cross + c2
        idx = jnp.argmin(dist, axis=1).astype(jnp.int32)
        ohb = (iota == idx[:, None]).astype(jnp.bfloat16)
        b0 = jnp.dot(ohb, by0_ref[d], preferred_element_type=jnp.float32)
        b1 = jnp.dot(ohb, by1_ref[d], preferred_element_type=jnp.float32)
        b2 = jnp.dot(ohb, by2_ref[d], preferred_element_type=jnp.float32)
        b3 = jnp.dot(ohb, by3_ref[d], preferred_element_type=jnp.float32)
        hi = (b3 * 256.0 + b2).astype(jnp.int32)
        lo = (b1 * 256.0 + b0).astype(jnp.int32)
        q = lax.bitcast_convert_type((hi << 16) | lo, jnp.float32)
        quant = quant + q
        r = r - q
        codes_ref[0, d, :] = idx
    vec_ref[0] = z + (quant - z)
    resid = z - quant
    loss_ref[...] = jnp.broadcast_to(jnp.sum(resid * resid) * scale,
                                     loss_ref.shape)


def _vq(lats, codebooks, bt):
    nl, n, emb = lats.shape
    depth, k, _ = codebooks.shape
    scale = 1.25 / (n * emb)
    w = lax.bitcast_convert_type(codebooks, jnp.int32)
    by = [((w >> (8 * i)) & 0xFF).astype(jnp.bfloat16) for i in range(4)]
    body = functools.partial(_vq_body, depth, k, scale)
    cb_spec = pl.BlockSpec((depth, k, emb), lambda l, t: (0, 0, 0))
    return pl.pallas_call(
        body,
        grid=(nl, n // bt),
        in_specs=[
            pl.BlockSpec((1, bt, emb), lambda l, t: (l, t, 0)),
            cb_spec, cb_spec, cb_spec, cb_spec, cb_spec,
        ],
        out_specs=[
            pl.BlockSpec((1, depth, bt), lambda l, t: (l, 0, t)),
            pl.BlockSpec((1, bt, emb), lambda l, t: (l, t, 0)),
            pl.BlockSpec((1, 1, 128), lambda l, t: (l, 0, t)),
        ],
        out_shape=[
            jax.ShapeDtypeStruct((nl, depth, n), jnp.int32),
            jax.ShapeDtypeStruct((nl, n, emb), jnp.float32),
            jax.ShapeDtypeStruct((nl, 1, (n // bt) * 128), jnp.float32),
        ],
        scratch_shapes=[pltpu.VMEM((depth, k), jnp.float32)],
    )(lats, codebooks, *by)


def kernel(modal_input_0, modal_input_1, modal_input_2, modal_input_3,
           modal_input_4, modal_input_5,
           proj_g_0, proj_b_0, proj_W_0, proj_bias_0,
           proj_g_1, proj_b_1, proj_W_1, proj_bias_1,
           proj_g_2, proj_b_2, proj_W_2, proj_bias_2,
           proj_g_3, proj_b_3, proj_W_3, proj_bias_3,
           proj_g_4, proj_b_4, proj_W_4, proj_bias_4,
           proj_g_5, proj_b_5, proj_W_5, proj_bias_5,
           dec_g_0, dec_b_0, dec_W1_0, dec_b1_0, dec_W2_0, dec_b2_0,
           dec_g_1, dec_b_1, dec_W1_1, dec_b1_1, dec_W2_1, dec_b2_1,
           dec_g_2, dec_b_2, dec_W1_2, dec_b1_2, dec_W2_2, dec_b2_2,
           dec_g_3, dec_b_3, dec_W1_3, dec_b1_3, dec_W2_3, dec_b2_3,
           codebooks):
    xs = [modal_input_0, modal_input_1, modal_input_2, modal_input_3,
          modal_input_4, modal_input_5]
    pg = [proj_g_0, proj_g_1, proj_g_2, proj_g_3, proj_g_4, proj_g_5]
    pb = [proj_b_0, proj_b_1, proj_b_2, proj_b_3, proj_b_4, proj_b_5]
    pw = [proj_W_0, proj_W_1, proj_W_2, proj_W_3, proj_W_4, proj_W_5]
    pbias = [proj_bias_0, proj_bias_1, proj_bias_2, proj_bias_3,
             proj_bias_4, proj_bias_5]
    n = xs[0].shape[0]
    bt = 512 if n % 512 == 0 else n

    modal_latents = _projectors(xs, pg, pb, pw, pbias, bt)
    latents = [modal_latents[i] for i in range(6)]

    dg = [dec_g_0, dec_g_1, dec_g_2, dec_g_3]
    db = [dec_b_0, dec_b_1, dec_b_2, dec_b_3]
    dw1 = [dec_W1_0, dec_W1_1, dec_W1_2, dec_W1_3]
    db1 = [dec_b1_0, dec_b1_1, dec_b1_2, dec_b1_3]
    dw2 = [dec_W2_0, dec_W2_1, dec_W2_2, dec_W2_3]
    db2 = [dec_b2_0, dec_b2_1, dec_b2_2, dec_b2_3]
    dec_in = [latents[:3], [latents[3]], [latents[4]], [latents[5]]]
    recons = [_decoder(dec_in[j], dg[j], db[j], dw1[j], db1[j],
                       dw2[j], db2[j], bt)
              for j in range(4)]

    latent_codes, latent_vectors, loss = _vq(modal_latents, codebooks, bt)
    vq_loss = jnp.sum(loss[:, 0, ::128])
    return (latent_codes, latent_vectors, vq_loss, modal_latents,
            recons[0], recons[1], recons[2], recons[3])


# speedup vs baseline: 1.7185x; 1.0311x over previous
import functools
import math

import jax
import jax.numpy as jnp
from jax import lax
from jax.experimental import pallas as pl
from jax.experimental.pallas import tpu as pltpu

_PREC = None


def _layernorm(x, g, b):
    m = jnp.mean(x, axis=-1, keepdims=True)
    v = jnp.mean((x - m) ** 2, axis=-1, keepdims=True)
    return (x - m) / jnp.sqrt(v + 1e-5) * g + b


def _proj_body(nmod, refs):
    o_ref = refs[-1]
    for i in range(nmod):
        x_ref, g_ref, b_ref, w_ref, bias_ref = refs[5 * i:5 * i + 5]
        h = _layernorm(x_ref[...], g_ref[...], b_ref[...])
        h = jnp.dot(h, w_ref[...], preferred_element_type=jnp.float32,
                    precision=_PREC) + bias_ref[...]
        o_ref[i] = h * jax.nn.sigmoid(h)


def _projectors(xs, gs, bs, ws, biases, bt):
    n = xs[0].shape[0]
    emb = ws[0].shape[1]
    nmod = len(xs)
    in_specs = []
    args = []
    for x, g, b, w, bias in zip(xs, gs, bs, ws, biases):
        d = x.shape[1]
        in_specs += [
            pl.BlockSpec((bt, d), lambda i: (i, 0)),
            pl.BlockSpec((1, d), lambda i: (0, 0)),
            pl.BlockSpec((1, d), lambda i: (0, 0)),
            pl.BlockSpec((d, emb), lambda i: (0, 0)),
            pl.BlockSpec((1, emb), lambda i: (0, 0)),
        ]
        args += [x, g.reshape(1, d), b.reshape(1, d), w, bias.reshape(1, emb)]
    return pl.pallas_call(
        functools.partial(lambda nm, *refs: _proj_body(nm, refs), nmod),
        grid=(n // bt,),
        in_specs=in_specs,
        out_specs=pl.BlockSpec((nmod, bt, emb), lambda i: (0, i, 0)),
        out_shape=jax.ShapeDtypeStruct((nmod, n, emb), jnp.float32),
    )(*args)


_INV_SQRT2 = 1.0 / math.sqrt(2.0)


def _dec_body(n_in, z_refs_and_w):
    (*z_refs, g_ref, b_ref, w1_ref, b1_ref, w2_ref, b2_ref, o_ref) = z_refs_and_w
    z = z_refs[0][...]
    for zr in z_refs[1:]:
        z = z + zr[...]
    h = _layernorm(z, g_ref[...], b_ref[...])
    h = jnp.dot(h, w1_ref[...], preferred_element_type=jnp.float32,
                precision=_PREC) + b1_ref[...]
    h = 0.5 * h * (1.0 + lax.erf(h * _INV_SQRT2))
    o_ref[...] = jnp.dot(h, w2_ref[...], preferred_element_type=jnp.float32,
                         precision=_PREC) + b2_ref[...]


def _decoder(zs, g, b, w1, b1, w2, b2, bt):
    n, emb = zs[0].shape
    hid = w1.shape[1]
    o = w2.shape[1]
    body = functools.partial(lambda *refs: _dec_body(len(zs), refs))
    return pl.pallas_call(
        body,
        grid=(n // bt,),
        in_specs=[pl.BlockSpec((bt, emb), lambda i: (i, 0)) for _ in zs] + [
            pl.BlockSpec((1, emb), lambda i: (0, 0)),
            pl.BlockSpec((1, emb), lambda i: (0, 0)),
            pl.BlockSpec((emb, hid), lambda i: (0, 0)),
            pl.BlockSpec((1, hid), lambda i: (0, 0)),
            pl.BlockSpec((hid, o), lambda i: (0, 0)),
            pl.BlockSpec((1, o), lambda i: (0, 0)),
        ],
        out_specs=pl.BlockSpec((bt, o), lambda i: (i, 0)),
        out_shape=jax.ShapeDtypeStruct((n, o), jnp.float32),
    )(*zs, g.reshape(1, emb), b.reshape(1, emb), w1, b1.reshape(1, hid),
      w2, b2.reshape(1, o))


def _vq_body(depth, k, scale, z_ref, cb_ref, by0_ref, by1_ref, by2_ref,
             by3_ref, codes_ref, vec_ref, loss_ref, c2_ref):
    z = z_ref[0]
    r = z
    quant = jnp.zeros_like(z)
    iota = lax.broadcasted_iota(jnp.int32, (z.shape[0], k), 1)

    first = (pl.program_id(0) == 0) & (pl.program_id(1) == 0)

    @pl.when(first)
    def _():
        c2_ref[...] = jnp.sum(cb_ref[...] * cb_ref[...], axis=2)

    for d in range(depth):
        cb = cb_ref[d]
        c2 = c2_ref[d][None, :]
        r2 = jnp.sum(r * r, axis=1, keepdims=True)
        cross = lax.dot_general(r, cb, (((1,), (1,)), ((), ())),
                                preferred_element_type=jnp.float32,
                                precision=_PREC)
        dist = r2 - 2.0 * cross + c2
        idx = jnp.argmin(dist, axis=1).astype(jnp.int32)
        ohb = (iota == idx[:, None]).astype(jnp.bfloat16)
        b0 = jnp.dot(ohb, by0_ref[d], preferred_element_type=jnp.float32)
        b1 = jnp.dot(ohb, by1_ref[d], preferred_element_type=jnp.float32)
        b2 = jnp.dot(ohb, by2_ref[d], preferred_element_type=jnp.float32)
        b3 = jnp.dot(ohb, by3_ref[d], preferred_element_type=jnp.float32)
        hi = (b3 * 256.0 + b2).astype(jnp.int32)
        lo = (b1 * 256.0 + b0).astype(jnp.int32)
        q = lax.bitcast_convert_type((hi << 16) | lo, jnp.float32)
        quant = quant + q
        r = r - q
        codes_ref[0, d, :] = idx
    vec_ref[0] = z + (quant - z)
    resid = z - quant
    loss_ref[...] = jnp.broadcast_to(jnp.sum(resid * resid) * scale,
                                     loss_ref.shape)


def _vq(lats, codebooks, bt):
    nl, n, emb = lats.shape
    depth, k, _ = codebooks.shape
    scale = 1.25 / (n * emb)
    w = lax.bitcast_convert_type(codebooks, jnp.int32)
    by = [((w >> (8 * i)) & 0xFF).astype(jnp.bfloat16) for i in range(4)]
    body = functools.partial(_vq_body, depth, k, scale)
    cb_spec = pl.BlockSpec((depth, k, emb), lambda l, t: (0, 0, 0))
    return pl.pallas_call(
        body,
        grid=(nl, n // bt),
        in_specs=[
            pl.BlockSpec((1, bt, emb), lambda l, t: (l, t, 0)),
            cb_spec, cb_spec, cb_spec, cb_spec, cb_spec,
        ],
        out_specs=[
            pl.BlockSpec((1, depth, bt), lambda l, t: (l, 0, t)),
            pl.BlockSpec((1, bt, emb), lambda l, t: (l, t, 0)),
            pl.BlockSpec((1, 1, 128), lambda l, t: (l, 0, t)),
        ],
        out_shape=[
            jax.ShapeDtypeStruct((nl, depth, n), jnp.int32),
            jax.ShapeDtypeStruct((nl, n, emb), jnp.float32),
            jax.ShapeDtypeStruct((nl, 1, (n // bt) * 128), jnp.float32),
        ],
        scratch_shapes=[pltpu.VMEM((depth, k), jnp.float32)],
    )(lats, codebooks, *by)


def kernel(modal_input_0, modal_input_1, modal_input_2, modal_input_3,
           modal_input_4, modal_input_5,
           proj_g_0, proj_b_0, proj_W_0, proj_bias_0,
           proj_g_1, proj_b_1, proj_W_1, proj_bias_1,
           proj_g_2, proj_b_2, proj_W_2, proj_bias_2,
           proj_g_3, proj_b_3, proj_W_3, proj_bias_3,
           proj_g_4, proj_b_4, proj_W_4, proj_bias_4,
           proj_g_5, proj_b_5, proj_W_5, proj_bias_5,
           dec_g_0, dec_b_0, dec_W1_0, dec_b1_0, dec_W2_0, dec_b2_0,
           dec_g_1, dec_b_1, dec_W1_1, dec_b1_1, dec_W2_1, dec_b2_1,
           dec_g_2, dec_b_2, dec_W1_2, dec_b1_2, dec_W2_2, dec_b2_2,
           dec_g_3, dec_b_3, dec_W1_3, dec_b1_3, dec_W2_3, dec_b2_3,
           codebooks):
    xs = [modal_input_0, modal_input_1, modal_input_2, modal_input_3,
          modal_input_4, modal_input_5]
    pg = [proj_g_0, proj_g_1, proj_g_2, proj_g_3, proj_g_4, proj_g_5]
    pb = [proj_b_0, proj_b_1, proj_b_2, proj_b_3, proj_b_4, proj_b_5]
    pw = [proj_W_0, proj_W_1, proj_W_2, proj_W_3, proj_W_4, proj_W_5]
    pbias = [proj_bias_0, proj_bias_1, proj_bias_2, proj_bias_3,
             proj_bias_4, proj_bias_5]
    n = xs[0].shape[0]
    bt = 512 if n % 512 == 0 else n
    bt2 = 1024 if n % 1024 == 0 else bt

    modal_latents = _projectors(xs, pg, pb, pw, pbias, bt)
    latents = [modal_latents[i] for i in range(6)]

    dg = [dec_g_0, dec_g_1, dec_g_2, dec_g_3]
    db = [dec_b_0, dec_b_1, dec_b_2, dec_b_3]
    dw1 = [dec_W1_0, dec_W1_1, dec_W1_2, dec_W1_3]
    db1 = [dec_b1_0, dec_b1_1, dec_b1_2, dec_b1_3]
    dw2 = [dec_W2_0, dec_W2_1, dec_W2_2, dec_W2_3]
    db2 = [dec_b2_0, dec_b2_1, dec_b2_2, dec_b2_3]
    dec_in = [latents[:3], [latents[3]], [latents[4]], [latents[5]]]
    recons = [_decoder(dec_in[j], dg[j], db[j], dw1[j], db1[j],
                       dw2[j], db2[j], bt2)
              for j in range(4)]

    latent_codes, latent_vectors, loss = _vq(modal_latents, codebooks, bt2)
    vq_loss = jnp.sum(loss[:, 0, ::128])
    return (latent_codes, latent_vectors, vq_loss, modal_latents,
            recons[0], recons[1], recons[2], recons[3])
